# baseline (device time: 66907 ns/iter reference)
import jax
import jax.numpy as jnp
from jax import lax
from jax.experimental import pallas as pl
from jax.experimental.pallas import tpu as pltpu

CW = 128
K = 16
GROUP = 8


def kernel(x, dy):
    m, d = x.shape
    _, f = dy.shape
    d_half = d // 2
    f_half = f // 2
    assert K * CW == f_half

    def body(x_ref, dy_ref, out_ref, xsend_ref, xrecv_ref,
             xsend_sems, xrecv_sems, ysend_sems, yrecv_sems):
        my_x = lax.axis_index("x")
        my_y = lax.axis_index("y")
        peer_x = 1 - my_x
        peer_y = 1 - my_y

        barrier = pltpu.get_barrier_semaphore()
        pl.semaphore_signal(barrier, inc=1, device_id=(peer_x, my_y),
                            device_id_type=pl.DeviceIdType.MESH)
        pl.semaphore_signal(barrier, inc=1, device_id=(my_x, peer_y),
                            device_id_type=pl.DeviceIdType.MESH)

        dims = (((0,), (0,)), ((), ()))
        x_peer = x_ref[:, pl.ds(peer_x * d_half, d_half)]

        rdma_x = []
        gw = GROUP * CW
        for g in range(K // GROUP):
            o = g * gw
            xsend_ref[:, o:o + gw] = lax.dot_general(
                x_peer, dy_ref[:, pl.ds(my_y * f_half + o, gw)], dims,
                preferred_element_type=jnp.float32)
            if g == 0:
                pl.semaphore_wait(barrier, 2)
            for j in range(GROUP):
                k = g * GROUP + j
                co = k * CW
                r = pltpu.make_async_remote_copy(
                    src_ref=xsend_ref.at[:, co:co + CW],
                    dst_ref=xrecv_ref.at[:, co:co + CW],
                    send_sem=xsend_sems.at[k], recv_sem=xrecv_sems.at[k],
                    device_id=(peer_x, my_y),
                    device_id_type=pl.DeviceIdType.MESH)
                r.start()
                rdma_x.append(r)

        out_ref[:, pl.ds(my_y * f_half, f_half)] = lax.dot_general(
            x_ref[:, pl.ds(my_x * d_half, d_half)],
            dy_ref[:, pl.ds(my_y * f_half, f_half)], dims,
            preferred_element_type=jnp.float32)

        rdma_y = []
        for k in range(K):
            co = k * CW
            rdma_x[k].wait_recv()
            sl = pl.ds(my_y * f_half + co, CW)
            out_ref[:, sl] = out_ref[:, sl] + xrecv_ref[:, co:co + CW]
            r = pltpu.make_async_remote_copy(
                src_ref=out_ref.at[:, sl],
                dst_ref=out_ref.at[:, sl],
                send_sem=ysend_sems.at[k], recv_sem=yrecv_sems.at[k],
                device_id=(my_x, peer_y),
                device_id_type=pl.DeviceIdType.MESH)
            r.start()
            rdma_y.append(r)

        for k in range(K):
            rdma_y[k].wait_recv()
        for k in range(K):
            rdma_x[k].wait_send()
            rdma_y[k].wait_send()

    return pl.pallas_call(
        body,
        out_shape=jax.ShapeDtypeStruct((d_half, f), jnp.float32),
        in_specs=[pl.BlockSpec(memory_space=pltpu.VMEM),
                  pl.BlockSpec(memory_space=pltpu.VMEM)],
        out_specs=pl.BlockSpec(memory_space=pltpu.VMEM),
        scratch_shapes=[
            pltpu.VMEM((d_half, f_half), jnp.float32),
            pltpu.VMEM((d_half, f_half), jnp.float32),
            pltpu.SemaphoreType.DMA((K,)),
            pltpu.SemaphoreType.DMA((K,)),
            pltpu.SemaphoreType.DMA((K,)),
            pltpu.SemaphoreType.DMA((K,)),
        ],
        compiler_params=pltpu.CompilerParams(collective_id=0),
    )(x, dy)


# device time: 63307 ns/iter; 1.0569x vs baseline; 1.0569x over previous
import jax
import jax.numpy as jnp
from jax import lax
from jax.experimental import pallas as pl
from jax.experimental.pallas import tpu as pltpu

CW = 128
K = 16
GROUP = 4


def kernel(x, dy):
    m, d = x.shape
    _, f = dy.shape
    d_half = d // 2
    f_half = f // 2
    assert K * CW == f_half

    def body(x_ref, dy_hbm, out_ref, dy_vmem, xsend_ref, xrecv_ref,
             xsend_sems, xrecv_sems, ysend_sems, yrecv_sems, dy_sems):
        my_x = lax.axis_index("x")
        my_y = lax.axis_index("y")
        peer_x = 1 - my_x
        peer_y = 1 - my_y
        half = f_half // 2

        cp0 = pltpu.make_async_copy(
            dy_hbm.at[:, pl.ds(my_y * f_half, half)],
            dy_vmem.at[:, 0:half], dy_sems.at[0])
        cp0.start()
        cp1 = pltpu.make_async_copy(
            dy_hbm.at[:, pl.ds(my_y * f_half + half, half)],
            dy_vmem.at[:, half:f_half], dy_sems.at[1])
        cp1.start()

        barrier = pltpu.get_barrier_semaphore()
        pl.semaphore_signal(barrier, inc=1, device_id=(peer_x, my_y),
                            device_id_type=pl.DeviceIdType.MESH)
        pl.semaphore_signal(barrier, inc=1, device_id=(my_x, peer_y),
                            device_id_type=pl.DeviceIdType.MESH)

        dims = (((0,), (0,)), ((), ()))
        x_peer = x_ref[:, pl.ds(peer_x * d_half, d_half)]

        rdma_x = []
        gw = GROUP * CW
        for g in range(K // GROUP):
            o = g * gw
            if o == 0:
                cp0.wait()
            if o == half:
                cp1.wait()
            xsend_ref[:, o:o + gw] = lax.dot_general(
                x_peer, dy_vmem[:, o:o + gw], dims,
                preferred_element_type=jnp.float32)
            if g == 0:
                pl.semaphore_wait(barrier, 2)
            for j in range(GROUP):
                k = g * GROUP + j
                co = k * CW
                r = pltpu.make_async_remote_copy(
                    src_ref=xsend_ref.at[:, co:co + CW],
                    dst_ref=xrecv_ref.at[:, co:co + CW],
                    send_sem=xsend_sems.at[k], recv_sem=xrecv_sems.at[k],
                    device_id=(peer_x, my_y),
                    device_id_type=pl.DeviceIdType.MESH)
                r.start()
                rdma_x.append(r)

        out_ref[:, pl.ds(my_y * f_half, f_half)] = lax.dot_general(
            x_ref[:, pl.ds(my_x * d_half, d_half)], dy_vmem[...], dims,
            preferred_element_type=jnp.float32)

        rdma_y = []
        for k in range(K):
            co = k * CW
            rdma_x[k].wait_recv()
            sl = pl.ds(my_y * f_half + co, CW)
            out_ref[:, sl] = out_ref[:, sl] + xrecv_ref[:, co:co + CW]
            r = pltpu.make_async_remote_copy(
                src_ref=out_ref.at[:, sl],
                dst_ref=out_ref.at[:, sl],
                send_sem=ysend_sems.at[k], recv_sem=yrecv_sems.at[k],
                device_id=(my_x, peer_y),
                device_id_type=pl.DeviceIdType.MESH)
            r.start()
            rdma_y.append(r)

        for k in range(K):
            rdma_y[k].wait_recv()
        for k in range(K):
            rdma_x[k].wait_send()
            rdma_y[k].wait_send()

    return pl.pallas_call(
        body,
        out_shape=jax.ShapeDtypeStruct((d_half, f), jnp.float32),
        in_specs=[pl.BlockSpec(memory_space=pltpu.VMEM),
                  pl.BlockSpec(memory_space=pl.ANY)],
        out_specs=pl.BlockSpec(memory_space=pltpu.VMEM),
        scratch_shapes=[
            pltpu.VMEM((m, f_half), jnp.float32),
            pltpu.VMEM((d_half, f_half), jnp.float32),
            pltpu.VMEM((d_half, f_half), jnp.float32),
            pltpu.SemaphoreType.DMA((K,)),
            pltpu.SemaphoreType.DMA((K,)),
            pltpu.SemaphoreType.DMA((K,)),
            pltpu.SemaphoreType.DMA((K,)),
            pltpu.SemaphoreType.DMA((2,)),
        ],
        compiler_params=pltpu.CompilerParams(collective_id=0),
    )(x, dy)


# device time: 62569 ns/iter; 1.0693x vs baseline; 1.0118x over previous
import jax
import jax.numpy as jnp
from jax import lax
from jax.experimental import pallas as pl
from jax.experimental.pallas import tpu as pltpu

CW = 128
K = 16
GROUP = 4


def kernel(x, dy):
    m, d = x.shape
    _, f = dy.shape
    d_half = d // 2
    f_half = f // 2
    assert K * CW == f_half

    def body(x_ref, dy_hbm, out_ref, dy_vmem, xsend_ref, xrecv_ref,
             xsend_sems, xrecv_sems, ysend_sems, yrecv_sems, dy_sems):
        my_x = lax.axis_index("x")
        my_y = lax.axis_index("y")
        peer_x = 1 - my_x
        peer_y = 1 - my_y
        gw = GROUP * CW
        n_groups = K // GROUP
        cps = []
        for g in range(n_groups):
            cp = pltpu.make_async_copy(
                dy_hbm.at[:, pl.ds(my_y * f_half + g * gw, gw)],
                dy_vmem.at[:, g * gw:(g + 1) * gw], dy_sems.at[g])
            cp.start()
            cps.append(cp)

        barrier = pltpu.get_barrier_semaphore()
        pl.semaphore_signal(barrier, inc=1, device_id=(peer_x, my_y),
                            device_id_type=pl.DeviceIdType.MESH)
        pl.semaphore_signal(barrier, inc=1, device_id=(my_x, peer_y),
                            device_id_type=pl.DeviceIdType.MESH)

        dims = (((0,), (0,)), ((), ()))
        x_peer = x_ref[:, pl.ds(peer_x * d_half, d_half)]

        rdma_x = []
        for g in range(K // GROUP):
            o = g * gw
            cps[g].wait()
            xsend_ref[:, o:o + gw] = lax.dot_general(
                x_peer, dy_vmem[:, o:o + gw], dims,
                preferred_element_type=jnp.float32)
            if g == 0:
                pl.semaphore_wait(barrier, 2)
            for j in range(GROUP):
                k = g * GROUP + j
                co = k * CW
                r = pltpu.make_async_remote_copy(
                    src_ref=xsend_ref.at[:, co:co + CW],
                    dst_ref=xrecv_ref.at[:, co:co + CW],
                    send_sem=xsend_sems.at[k], recv_sem=xrecv_sems.at[k],
                    device_id=(peer_x, my_y),
                    device_id_type=pl.DeviceIdType.MESH)
                r.start()
                rdma_x.append(r)

        out_ref[:, pl.ds(my_y * f_half, f_half)] = lax.dot_general(
            x_ref[:, pl.ds(my_x * d_half, d_half)], dy_vmem[...], dims,
            preferred_element_type=jnp.float32)

        rdma_y = []
        for k in range(K):
            co = k * CW
            rdma_x[k].wait_recv()
            sl = pl.ds(my_y * f_half + co, CW)
            out_ref[:, sl] = out_ref[:, sl] + xrecv_ref[:, co:co + CW]
            r = pltpu.make_async_remote_copy(
                src_ref=out_ref.at[:, sl],
                dst_ref=out_ref.at[:, sl],
                send_sem=ysend_sems.at[k], recv_sem=yrecv_sems.at[k],
                device_id=(my_x, peer_y),
                device_id_type=pl.DeviceIdType.MESH)
            r.start()
            rdma_y.append(r)

        for k in range(K):
            rdma_y[k].wait_recv()
        for k in range(K):
            rdma_x[k].wait_send()
            rdma_y[k].wait_send()

    return pl.pallas_call(
        body,
        out_shape=jax.ShapeDtypeStruct((d_half, f), jnp.float32),
        in_specs=[pl.BlockSpec(memory_space=pltpu.VMEM),
                  pl.BlockSpec(memory_space=pl.ANY)],
        out_specs=pl.BlockSpec(memory_space=pltpu.VMEM),
        scratch_shapes=[
            pltpu.VMEM((m, f_half), jnp.float32),
            pltpu.VMEM((d_half, f_half), jnp.float32),
            pltpu.VMEM((d_half, f_half), jnp.float32),
            pltpu.SemaphoreType.DMA((K,)),
            pltpu.SemaphoreType.DMA((K,)),
            pltpu.SemaphoreType.DMA((K,)),
            pltpu.SemaphoreType.DMA((K,)),
            pltpu.SemaphoreType.DMA((K // GROUP,)),
        ],
        compiler_params=pltpu.CompilerParams(collective_id=0),
    )(x, dy)
